# baseline (speedup 1.0000x reference)
import functools

import jax
import jax.numpy as jnp
from jax import lax
from jax.experimental import pallas as pl
from jax.experimental.pallas import tpu as pltpu
from jax.experimental.pallas import tpu_sc as plsc

F32 = jnp.float32
_INF = float("inf")


def _knn_body(K, S, BT, r2, ptx, pty, ptz, psx, psy, psz, idx_ref, msk_ref,
              d2_ref):
    ax, ay, az = ptx[...], pty[...], ptz[...]
    bx, by, bz = psx[...], psy[...], psz[...]
    dx = ax - bx
    dy = ay - by
    dz = az - bz
    d2_ref[...] = dx * dx + dy * dy + dz * dz
    iota_s = lax.broadcasted_iota(jnp.int32, (BT, S), 1)
    lane_k = lax.broadcasted_iota(jnp.int32, (BT, K), 1)

    UNR = 8

    def step(tq, carry):
        acc_i, acc_v = carry
        d2 = d2_ref[...]
        for u in range(UNR):
            t = tq * UNR + u
            m = jnp.min(d2, axis=1, keepdims=True)
            sel = jnp.min(jnp.where(d2 == m, iota_s, S), axis=1,
                          keepdims=True)
            acc_i = jnp.where(lane_k == t, sel, acc_i)
            acc_v = jnp.where(lane_k == t, m, acc_v)
            d2 = jnp.where(iota_s == sel, _INF, d2)
        d2_ref[...] = d2
        return acc_i, acc_v

    acc_i = jnp.zeros((BT, K), jnp.int32)
    acc_v = jnp.full((BT, K), _INF, F32)
    acc_i, acc_v = lax.fori_loop(0, K // UNR, step, (acc_i, acc_v))
    idx_ref[...] = acc_i
    msk_ref[...] = (acc_v <= r2).astype(F32)


def _knn(pos_t, pos_s, K, r2, BT):
    NT = pos_t.shape[0]
    NS = pos_s.shape[0]
    tcols = [pos_t[:, c:c + 1] for c in range(3)]
    srows = [pos_s[:, c].reshape(1, NS) for c in range(3)]
    grid = NT // BT
    t_spec = pl.BlockSpec((BT, 1), lambda b: (b, 0))
    s_spec = pl.BlockSpec((1, NS), lambda b: (0, 0))
    o_spec = pl.BlockSpec((BT, K), lambda b: (b, 0))
    return pl.pallas_call(
        functools.partial(_knn_body, K, NS, BT, r2),
        grid=(grid,),
        in_specs=[t_spec] * 3 + [s_spec] * 3,
        out_specs=[o_spec, o_spec],
        out_shape=[jax.ShapeDtypeStruct((NT, K), jnp.int32),
                   jax.ShapeDtypeStruct((NT, K), F32)],
        scratch_shapes=[pltpu.VMEM((BT, NS), F32)],
    )(*tcols, *srows)


def _bdot(x, a):
    return jnp.dot(x.astype(jnp.bfloat16), a.astype(jnp.bfloat16),
                   preferred_element_type=F32)


def _matmul_body(x, a, c, out):
    out[...] = _bdot(x[...], a[...]) + c[...]


def _matmul(X, A, c):
    N, D = X.shape[0], A.shape[1]
    return pl.pallas_call(
        _matmul_body,
        out_shape=jax.ShapeDtypeStruct((N, D), F32),
    )(X, A, c)


def _dpos_term(pg, ct, wc, TB, K):
    dp = (pg.reshape(TB, K, 16) - ct[:, None, :]).reshape(TB * K, 16)
    dp = dp.astype(jnp.bfloat16).astype(F32)
    w = wc.astype(jnp.bfloat16).astype(F32)
    return (dp[:, 0:1] * w[0:1, :] + dp[:, 1:2] * w[1:2, :]
            + dp[:, 2:3] * w[2:3, :])


def _gather_rows(table, idx_flat):
    B = idx_flat.shape[0]
    S, D = table.shape
    NW = 32
    bpw = B // NW
    ch = min(bpw, 128)
    nchunk = bpw // ch
    nbuf = 4 if nchunk % 4 == 0 else 1
    idx_r = idx_flat.reshape(NW, nchunk, ch)
    mesh = plsc.VectorSubcoreMesh(core_axis_name="c", subcore_axis_name="s")

    @functools.partial(
        pl.kernel,
        out_type=jax.ShapeDtypeStruct((B, D), F32),
        mesh=mesh,
        scratch_types=[
            pltpu.VMEM((nchunk, ch), jnp.int32),
        ] + [pltpu.VMEM((ch, D), F32) for _ in range(nbuf)]
        + [pltpu.SemaphoreType.DMA for _ in range(2 * nbuf)],
    )
    def gk(idx_hbm, table_hbm, out_hbm, idx_v, *bufs_sems):
        rows = bufs_sems[:nbuf]
        gsem = bufs_sems[nbuf:2 * nbuf]
        ssem = bufs_sems[2 * nbuf:]
        w = lax.axis_index("s") * 2 + lax.axis_index("c")
        base = w * bpw
        pltpu.sync_copy(idx_hbm.at[w], idx_v)

        if nbuf == 1:
            def body(j, carry):
                pltpu.async_copy(table_hbm.at[idx_v.at[j]], rows[0],
                                 gsem[0]).wait()
                pltpu.sync_copy(rows[0], out_hbm.at[pl.ds(base + j * ch, ch)])
                return carry

            lax.fori_loop(0, nchunk, body, 0)
            return

        for b in range(nbuf):
            pltpu.async_copy(table_hbm.at[idx_v.at[b]], rows[b], gsem[b])

        def block(jb, carry):
            jbase = jb * nbuf
            outs = []
            for b in range(nbuf):
                pltpu.make_async_copy(table_hbm.at[pl.ds(0, ch)], rows[b],
                                      gsem[b]).wait()
                cp = pltpu.async_copy(
                    rows[b], out_hbm.at[pl.ds(base + (jbase + b) * ch, ch)],
                    ssem[b])
                outs.append(cp)
            for b in range(nbuf):
                @pl.when(jbase + nbuf < nchunk)
                def _(b=b):
                    outs[b].wait()
                    pltpu.async_copy(table_hbm.at[idx_v.at[jbase + nbuf + b]],
                                     rows[b], gsem[b])
            return carry

        lax.fori_loop(0, nchunk // nbuf, block, 0)
        for b in range(nbuf):
            pltpu.make_async_copy(
                rows[b], out_hbm.at[pl.ds(base, ch)], ssem[b]).wait()

    return gk(idx_r, table)


def _stats_a_body(TB, K, g_ref, u_ref, pg_ref, ct_ref, wc_ref, mf_ref,
                  st_ref):
    @pl.when(pl.program_id(0) == 0)
    def _():
        st_ref[...] = jnp.zeros_like(st_ref)

    BR = TB * K
    y = (g_ref[...].reshape(TB, K, 128) + u_ref[...][:, None, :])
    y = y.reshape(BR, 128) + _dpos_term(pg_ref[...][:, :16],
                                        ct_ref[...][:, :16], wc_ref[...],
                                        TB, K)
    mf = mf_ref[...]
    ym = y * mf
    st_ref[0:1, :] += jnp.sum(ym, axis=0, keepdims=True)
    st_ref[1:2, :] += jnp.sum(ym * y, axis=0, keepdims=True)
    st_ref[2:3, :] += jnp.zeros((1, 128), F32) + jnp.sum(mf)


def _norm(y, st, g, be):
    cnt = jnp.maximum(st[2, 0], 1.0)
    mean = st[0:1, :] / cnt
    var = st[1:2, :] / cnt - mean * mean
    return jnp.maximum((y - mean) / jnp.sqrt(var + 1e-5) * g + be, 0.0)


def _l2_a_body(TB, K, g_ref, u_ref, pg_ref, ct_ref, wc_ref, mf_ref, st_ref,
               ga_ref, be_ref, w_ref, b_ref, y2_ref, st2_ref):
    @pl.when(pl.program_id(0) == 0)
    def _():
        st2_ref[...] = jnp.zeros_like(st2_ref)

    BR = TB * K
    y1 = (g_ref[...].reshape(TB, K, 128) + u_ref[...][:, None, :])
    y1 = y1.reshape(BR, 128) + _dpos_term(pg_ref[...][:, :16],
                                          ct_ref[...][:, :16], wc_ref[...],
                                          TB, K)
    h = _norm(y1, st_ref[...], ga_ref[...], be_ref[...])
    y2 = _bdot(h, w_ref[...]) + b_ref[...]
    y2_ref[...] = y2
    mf = mf_ref[...]
    ym = y2 * mf
    st2_ref[0:1, :] += jnp.sum(ym, axis=0, keepdims=True)
    st2_ref[1:2, :] += jnp.sum(ym * y2, axis=0, keepdims=True)
    st2_ref[2:3, :] += jnp.zeros((1, 128), F32) + jnp.sum(mf)


def _l3_a_body(y_ref, mf_ref, st_ref, ga_ref, be_ref, w_ref, b_ref, y3_ref,
               st3_ref):
    @pl.when(pl.program_id(0) == 0)
    def _():
        st3_ref[...] = jnp.zeros_like(st3_ref)

    h = _norm(y_ref[...], st_ref[...], ga_ref[...], be_ref[...])
    y3 = _bdot(h, w_ref[...]) + b_ref[...]
    y3_ref[...] = y3
    mf = mf_ref[...]
    ym = y3 * mf
    st3_ref[0:1, :] += jnp.sum(ym, axis=0, keepdims=True)
    st3_ref[1:2, :] += jnp.sum(ym * y3, axis=0, keepdims=True)
    st3_ref[2:3, :] += jnp.zeros((1, 128), F32) + jnp.sum(mf)


def _pool_a_body(TB, K, y_ref, st_ref, ga_ref, be_ref, m_ref, fe_ref):
    h = _norm(y_ref[...], st_ref[...], ga_ref[...], be_ref[...])
    h3 = h.reshape(TB, K, 128)
    m = m_ref[...]
    hm = jnp.where(m[:, :, None] > 0, h3, -_INF)
    pooled = jnp.max(hm, axis=1)
    anyv = jnp.max(m, axis=1, keepdims=True) > 0
    fe_ref[...] = jnp.where(anyv, pooled, 0.0)


def _stage_a_mlp(G, U, Pg, ct, wc, mask, params):
    (_, _, g1, be1), (W2, b2, g2, be2), (W3, b3, g3, be3) = params
    NT, K = mask.shape
    TB = 128
    BR = TB * K
    nblk = NT // TB
    mf = mask.reshape(NT * K, 1)
    g_spec = pl.BlockSpec((BR, 128), lambda b: (b, 0))
    u_spec = pl.BlockSpec((TB, 128), lambda b: (b, 0))
    pg_spec = pl.BlockSpec((BR, 128), lambda b: (b, 0))
    ct_spec = pl.BlockSpec((TB, 128), lambda b: (b, 0))
    wc_spec = pl.BlockSpec((3, 128), lambda b: (0, 0))
    mf_spec = pl.BlockSpec((BR, 1), lambda b: (b, 0))
    m_spec = pl.BlockSpec((TB, K), lambda b: (b, 0))
    st_spec = pl.BlockSpec((8, 128), lambda b: (0, 0))
    w_spec = pl.BlockSpec((128, 128), lambda b: (0, 0))
    r_spec = pl.BlockSpec((1, 128), lambda b: (0, 0))
    st_shape = jax.ShapeDtypeStruct((8, 128), F32)
    y_shape = jax.ShapeDtypeStruct((NT * K, 128), F32)

    st1 = pl.pallas_call(
        functools.partial(_stats_a_body, TB, K),
        grid=(nblk,),
        in_specs=[g_spec, u_spec, pg_spec, ct_spec, wc_spec, mf_spec],
        out_specs=st_spec,
        out_shape=st_shape,
    )(G, U, Pg, ct, wc, mf)

    y2, st2 = pl.pallas_call(
        functools.partial(_l2_a_body, TB, K),
        grid=(nblk,),
        in_specs=[g_spec, u_spec, pg_spec, ct_spec, wc_spec, mf_spec, st_spec,
                  r_spec, r_spec, w_spec, r_spec],
        out_specs=[g_spec, st_spec],
        out_shape=[y_shape, st_shape],
    )(G, U, Pg, ct, wc, mf, st1, g1.reshape(1, -1), be1.reshape(1, -1), W2.T,
      b2.reshape(1, -1))

    y3, st3 = pl.pallas_call(
        _l3_a_body,
        grid=(nblk,),
        in_specs=[g_spec, mf_spec, st_spec, r_spec, r_spec, w_spec, r_spec],
        out_specs=[g_spec, st_spec],
        out_shape=[y_shape, st_shape],
    )(y2, mf, st2, g2.reshape(1, -1), be2.reshape(1, -1), W3.T,
      b3.reshape(1, -1))

    fe = pl.pallas_call(
        functools.partial(_pool_a_body, TB, K),
        grid=(nblk,),
        in_specs=[g_spec, st_spec, r_spec, r_spec, m_spec],
        out_specs=u_spec,
        out_shape=jax.ShapeDtypeStruct((NT, 128), F32),
    )(y3, st3, g3.reshape(1, -1), be3.reshape(1, -1), mask)
    return fe


def _mlp_small_body(NT, K, D1, D2, D3, g_ref, pg_ref, ct_ref, wc_ref, b1_ref,
                    m_ref, mf_ref, ga1, be1, w2, b2, ga2, be2, w3, b3, ga3,
                    be3, fe_ref):
    m = m_ref[...]
    mf = mf_ref[...]
    cnt = jnp.maximum(jnp.sum(mf), 1.0)

    def bn(y, ga, be):
        ym = y * mf
        mean = jnp.sum(ym, axis=0, keepdims=True) / cnt
        var = jnp.sum(ym * y, axis=0, keepdims=True) / cnt - mean * mean
        return jnp.maximum((y - mean) / jnp.sqrt(var + 1e-5) * ga[...]
                           + be[...], 0.0)

    y1 = (g_ref[...] + _dpos_term(pg_ref[...][:, :16], ct_ref[...][:, :16],
                                  wc_ref[...], NT, K) + b1_ref[...])
    h1 = bn(y1, ga1, be1)
    y2 = _bdot(h1, w2[...]) + b2[...]
    h2 = bn(y2, ga2, be2)
    y3 = _bdot(h2, w3[...]) + b3[...]
    h3 = bn(y3, ga3, be3).reshape(NT, K, D3)
    hm = jnp.where(m[:, :, None] > 0, h3, -_INF)
    pooled = jnp.max(hm, axis=1)
    anyv = jnp.max(m, axis=1, keepdims=True) > 0
    fe_ref[...] = jnp.where(anyv, pooled, 0.0)


def _mlp_small(G, Pg, ct, wc, b1, mask, params):
    (_, _, g1, be1), (W2, b2, g2, be2), (W3, b3, g3, be3) = params
    NT, K = mask.shape
    D1 = G.shape[1]
    D2 = W2.shape[0]
    D3 = W3.shape[0]
    return pl.pallas_call(
        functools.partial(_mlp_small_body, NT, K, D1, D2, D3),
        out_shape=jax.ShapeDtypeStruct((NT, D3), F32),
    )(G, Pg, ct, wc, b1.reshape(1, -1), mask, mask.reshape(NT * K, 1),
      g1.reshape(1, -1), be1.reshape(1, -1), W2.T,
      b2.reshape(1, -1), g2.reshape(1, -1), be2.reshape(1, -1), W3.T,
      b3.reshape(1, -1), g3.reshape(1, -1), be3.reshape(1, -1))


def _fps_body(N, M, R, Mr, px_ref, py_ref, pz_ref, b_ref, cpx_ref, cpy_ref,
              cpz_ref, cb_ref):
    px, py, pz = px_ref[...], py_ref[...], pz_ref[...]
    bv = b_ref[...]
    iota = (lax.broadcasted_iota(jnp.int32, (R, 128), 0) * 128
            + lax.broadcasted_iota(jnp.int32, (R, 128), 1))
    iom = (lax.broadcasted_iota(jnp.int32, (Mr, 128), 0) * 128
           + lax.broadcasted_iota(jnp.int32, (Mr, 128), 1))
    px0, py0, pz0 = px[0, 0], py[0, 0], pz[0, 0]
    d0 = (px - px0) ** 2 + (py - py0) ** 2 + (pz - pz0) ** 2
    first = iom == 0
    apx = jnp.where(first, px0, 0.0)
    apy = jnp.where(first, py0, 0.0)
    apz = jnp.where(first, pz0, 0.0)
    ab = jnp.where(first, bv[0, 0], 0)

    def step(i, carry):
        dists, apx, apy, apz, ab = carry
        mx = jnp.max(dists)
        nxt = jnp.min(jnp.where(dists == mx, iota, N))
        selm = iota == nxt
        sx = jnp.sum(jnp.where(selm, px, 0.0))
        sy = jnp.sum(jnp.where(selm, py, 0.0))
        sz = jnp.sum(jnp.where(selm, pz, 0.0))
        sb = jnp.sum(jnp.where(selm, bv, 0))
        d = (px - sx) ** 2 + (py - sy) ** 2 + (pz - sz) ** 2
        dists = jnp.minimum(dists, d)
        hit = iom == i
        apx = jnp.where(hit, sx, apx)
        apy = jnp.where(hit, sy, apy)
        apz = jnp.where(hit, sz, apz)
        ab = jnp.where(hit, sb, ab)
        return dists, apx, apy, apz, ab

    _, apx, apy, apz, ab = lax.fori_loop(1, M, step, (d0, apx, apy, apz, ab))
    cpx_ref[...] = apx
    cpy_ref[...] = apy
    cpz_ref[...] = apz
    cb_ref[...] = ab


def _fps(pos8, b16, M):
    N = pos8.shape[0]
    R = N // 128
    Mr = M // 128
    pcs = [pos8[:, c].reshape(R, 128) for c in range(3)]
    o = jax.ShapeDtypeStruct((Mr, 128), F32)
    return pl.pallas_call(
        functools.partial(_fps_body, N, M, R, Mr),
        out_shape=[o, o, o, jax.ShapeDtypeStruct((Mr, 128), jnp.int32)],
    )(*pcs, b16)


def kernel(f1, pos1, batch1, f2, pos2, batch2, params_fe, params_sc1,
           params_sc2):
    N = f1.shape[0]
    pos1p = jnp.pad(pos1, ((0, 0), (0, 5)))
    pos2p = jnp.pad(pos2, ((0, 0), (0, 5)))
    pos1t = jnp.pad(pos1, ((0, 0), (0, 125)))
    pos2t = jnp.pad(pos2, ((0, 0), (0, 125)))

    W1, b1, _, _ = params_fe[0]
    WaT = W1[:, :128].T
    WbT = W1[:, 128:256].T
    WcT = W1[:, 256:259].T
    zrow = jnp.zeros((1, 128), F32)

    idx1, m1 = _knn(pos1p, pos2p, 64, 25.0, 256)
    U1 = _matmul(f1, WaT, b1.reshape(1, -1))
    V1 = _matmul(f2, WbT, zrow)
    G1 = _gather_rows(V1, idx1.reshape(-1))
    P1 = _gather_rows(pos2t, idx1.reshape(-1))
    fe1 = _stage_a_mlp(G1, U1, P1, pos1t, WcT, m1, params_fe)

    W1s, b1s, _, _ = params_sc1[0]
    WfT2 = W1s[:, :128].T
    WpT2 = W1s[:, 128:131].T
    cx2, cy2, cz2, cb2 = _fps(pos1p, batch1.reshape(N // 128, 128), N // 4)
    cpos2 = jnp.stack([cx2.reshape(-1), cy2.reshape(-1), cz2.reshape(-1)],
                      axis=1)
    cpos2p = jnp.pad(cpos2, ((0, 0), (0, 5)))
    cpos2t = jnp.pad(cpos2, ((0, 0), (0, 125)))
    idx2, m2 = _knn(cpos2p, pos1p, 8, 4.0, N // 4)
    V2 = _matmul(fe1, WfT2, zrow)
    G2 = _gather_rows(V2, idx2.reshape(-1))
    P2 = _gather_rows(pos1t, idx2.reshape(-1))
    fe2 = _mlp_small(G2, P2, cpos2t, WpT2, b1s, m2, params_sc1)

    W1t, b1t, _, _ = params_sc2[0]
    WfT3 = W1t[:, :256].T
    WpT3 = W1t[:, 256:259].T
    cx3, cy3, cz3, cb3 = _fps(cpos2p, cb2.reshape(N // 512, 128), N // 16)
    cpos3 = jnp.stack([cx3.reshape(-1), cy3.reshape(-1), cz3.reshape(-1)],
                      axis=1)
    cpos3p = jnp.pad(cpos3, ((0, 0), (0, 5)))
    cpos3t = jnp.pad(cpos3, ((0, 0), (0, 125)))
    idx3, m3 = _knn(cpos3p, cpos2p, 8, 16.0, N // 16)
    V3 = _matmul(fe2, WfT3, jnp.zeros((1, 256), F32))
    G3 = _gather_rows(V3, idx3.reshape(-1))
    P3 = _gather_rows(cpos2t, idx3.reshape(-1))
    fe3 = _mlp_small(G3, P3, cpos3t, WpT3, b1t, m3, params_sc2)

    return ((fe1, pos1, batch1),
            (fe2, cpos2, cb2.reshape(-1)),
            (fe3, cpos3, cb3.reshape(-1)))


# speedup vs baseline: 1.0350x; 1.0350x over previous
import functools

import jax
import jax.numpy as jnp
from jax import lax
from jax.experimental import pallas as pl
from jax.experimental.pallas import tpu as pltpu
from jax.experimental.pallas import tpu_sc as plsc

F32 = jnp.float32
_INF = float("inf")


def _knn_body(K, S, BT, r2, ptx, pty, ptz, psx, psy, psz, idx_ref, msk_ref,
              d2_ref):
    ax, ay, az = ptx[...], pty[...], ptz[...]
    bx, by, bz = psx[...], psy[...], psz[...]
    dx = ax - bx
    dy = ay - by
    dz = az - bz
    d2_ref[...] = dx * dx + dy * dy + dz * dz
    iota_s = lax.broadcasted_iota(jnp.int32, (BT, S), 1)
    lane_k = lax.broadcasted_iota(jnp.int32, (BT, K), 1)

    UNR = 8

    def step(tq, carry):
        acc_i, acc_v = carry
        d2 = d2_ref[...]
        for u in range(UNR):
            t = tq * UNR + u
            m = jnp.min(d2, axis=1, keepdims=True)
            sel = jnp.min(jnp.where(d2 == m, iota_s, S), axis=1,
                          keepdims=True)
            acc_i = jnp.where(lane_k == t, sel, acc_i)
            acc_v = jnp.where(lane_k == t, m, acc_v)
            d2 = jnp.where(iota_s == sel, _INF, d2)
        d2_ref[...] = d2
        return acc_i, acc_v

    acc_i = jnp.zeros((BT, K), jnp.int32)
    acc_v = jnp.full((BT, K), _INF, F32)
    acc_i, acc_v = lax.fori_loop(0, K // UNR, step, (acc_i, acc_v))
    idx_ref[...] = acc_i
    msk_ref[...] = (acc_v <= r2).astype(F32)


def _knn(pos_t, pos_s, K, r2, BT):
    NT = pos_t.shape[0]
    NS = pos_s.shape[0]
    tcols = [pos_t[:, c:c + 1] for c in range(3)]
    srows = [pos_s[:, c].reshape(1, NS) for c in range(3)]
    grid = NT // BT
    t_spec = pl.BlockSpec((BT, 1), lambda b: (b, 0))
    s_spec = pl.BlockSpec((1, NS), lambda b: (0, 0))
    o_spec = pl.BlockSpec((BT, K), lambda b: (b, 0))
    return pl.pallas_call(
        functools.partial(_knn_body, K, NS, BT, r2),
        grid=(grid,),
        in_specs=[t_spec] * 3 + [s_spec] * 3,
        out_specs=[o_spec, o_spec],
        out_shape=[jax.ShapeDtypeStruct((NT, K), jnp.int32),
                   jax.ShapeDtypeStruct((NT, K), F32)],
        scratch_shapes=[pltpu.VMEM((BT, NS), F32)],
    )(*tcols, *srows)


def _bdot(x, a):
    return jnp.dot(x.astype(jnp.bfloat16), a.astype(jnp.bfloat16),
                   preferred_element_type=F32)


def _matmul_body(x, a, c, out):
    out[...] = _bdot(x[...], a[...]) + c[...]


def _matmul(X, A, c):
    N, D = X.shape[0], A.shape[1]
    return pl.pallas_call(
        _matmul_body,
        out_shape=jax.ShapeDtypeStruct((N, D), F32),
    )(X, A, c)


def _dpos_term(pg, ct, wc, TB, K):
    dp = (pg.reshape(TB, K, 16) - ct[:, None, :]).reshape(TB * K, 16)
    dp = dp.astype(jnp.bfloat16).astype(F32)
    w = wc.astype(jnp.bfloat16).astype(F32)
    return (dp[:, 0:1] * w[0:1, :] + dp[:, 1:2] * w[1:2, :]
            + dp[:, 2:3] * w[2:3, :])


def _gather_rows(table, idx_flat):
    B = idx_flat.shape[0]
    S, D = table.shape
    NW = 32
    bpw = B // NW
    ch = min(bpw, 128)
    nchunk = bpw // ch
    nbuf = 4 if nchunk % 4 == 0 else 1
    idx_r = idx_flat.reshape(NW, nchunk, ch)
    mesh = plsc.VectorSubcoreMesh(core_axis_name="c", subcore_axis_name="s")

    @functools.partial(
        pl.kernel,
        out_type=jax.ShapeDtypeStruct((B, D), F32),
        mesh=mesh,
        scratch_types=[
            pltpu.VMEM((nchunk, ch), jnp.int32),
        ] + [pltpu.VMEM((ch, D), F32) for _ in range(nbuf)]
        + [pltpu.SemaphoreType.DMA for _ in range(2 * nbuf)],
    )
    def gk(idx_hbm, table_hbm, out_hbm, idx_v, *bufs_sems):
        rows = bufs_sems[:nbuf]
        gsem = bufs_sems[nbuf:2 * nbuf]
        ssem = bufs_sems[2 * nbuf:]
        w = lax.axis_index("s") * 2 + lax.axis_index("c")
        base = w * bpw
        pltpu.sync_copy(idx_hbm.at[w], idx_v)

        if nbuf == 1:
            def body(j, carry):
                pltpu.async_copy(table_hbm.at[idx_v.at[j]], rows[0],
                                 gsem[0]).wait()
                pltpu.sync_copy(rows[0], out_hbm.at[pl.ds(base + j * ch, ch)])
                return carry

            lax.fori_loop(0, nchunk, body, 0)
            return

        for b in range(nbuf):
            pltpu.async_copy(table_hbm.at[idx_v.at[b]], rows[b], gsem[b])

        def block(jb, carry):
            jbase = jb * nbuf
            outs = []
            for b in range(nbuf):
                pltpu.make_async_copy(table_hbm.at[pl.ds(0, ch)], rows[b],
                                      gsem[b]).wait()
                cp = pltpu.async_copy(
                    rows[b], out_hbm.at[pl.ds(base + (jbase + b) * ch, ch)],
                    ssem[b])
                outs.append(cp)
            for b in range(nbuf):
                @pl.when(jbase + nbuf < nchunk)
                def _(b=b):
                    outs[b].wait()
                    pltpu.async_copy(table_hbm.at[idx_v.at[jbase + nbuf + b]],
                                     rows[b], gsem[b])
            return carry

        lax.fori_loop(0, nchunk // nbuf, block, 0)
        for b in range(nbuf):
            pltpu.make_async_copy(
                rows[b], out_hbm.at[pl.ds(base, ch)], ssem[b]).wait()

    return gk(idx_r, table)


def _stats_a_body(TB, K, g_ref, u_ref, pg_ref, ct_ref, wc_ref, mf_ref,
                  st_ref):
    @pl.when(pl.program_id(0) == 0)
    def _():
        st_ref[...] = jnp.zeros_like(st_ref)

    BR = TB * K
    y = (g_ref[...].reshape(TB, K, 128) + u_ref[...][:, None, :])
    y = y.reshape(BR, 128) + _dpos_term(pg_ref[...][:, :16],
                                        ct_ref[...][:, :16], wc_ref[...],
                                        TB, K)
    mf = mf_ref[...]
    ym = y * mf
    st_ref[0:1, :] += jnp.sum(ym, axis=0, keepdims=True)
    st_ref[1:2, :] += jnp.sum(ym * y, axis=0, keepdims=True)
    st_ref[2:3, :] += jnp.zeros((1, 128), F32) + jnp.sum(mf)


def _norm(y, st, g, be):
    cnt = jnp.maximum(st[2, 0], 1.0)
    mean = st[0:1, :] / cnt
    var = st[1:2, :] / cnt - mean * mean
    return jnp.maximum((y - mean) / jnp.sqrt(var + 1e-5) * g + be, 0.0)


def _l2_a_body(TB, K, g_ref, u_ref, pg_ref, ct_ref, wc_ref, mf_ref, st_ref,
               ga_ref, be_ref, w_ref, b_ref, y2_ref, st2_ref):
    @pl.when(pl.program_id(0) == 0)
    def _():
        st2_ref[...] = jnp.zeros_like(st2_ref)

    BR = TB * K
    y1 = (g_ref[...].reshape(TB, K, 128) + u_ref[...][:, None, :])
    y1 = y1.reshape(BR, 128) + _dpos_term(pg_ref[...][:, :16],
                                          ct_ref[...][:, :16], wc_ref[...],
                                          TB, K)
    h = _norm(y1, st_ref[...], ga_ref[...], be_ref[...])
    y2 = _bdot(h, w_ref[...]) + b_ref[...]
    y2_ref[...] = y2
    mf = mf_ref[...]
    ym = y2 * mf
    st2_ref[0:1, :] += jnp.sum(ym, axis=0, keepdims=True)
    st2_ref[1:2, :] += jnp.sum(ym * y2, axis=0, keepdims=True)
    st2_ref[2:3, :] += jnp.zeros((1, 128), F32) + jnp.sum(mf)


def _l3_a_body(TB, K, y_ref, mf_ref, m_ref, st_ref, ga_ref, be_ref, w_ref,
               b_ref, hi_ref, lo_ref, st3_ref):
    @pl.when(pl.program_id(0) == 0)
    def _():
        st3_ref[...] = jnp.zeros_like(st3_ref)

    h = _norm(y_ref[...], st_ref[...], ga_ref[...], be_ref[...])
    y3 = _bdot(h, w_ref[...]) + b_ref[...]
    mf = mf_ref[...]
    ym = y3 * mf
    st3_ref[0:1, :] += jnp.sum(ym, axis=0, keepdims=True)
    st3_ref[1:2, :] += jnp.sum(ym * y3, axis=0, keepdims=True)
    st3_ref[2:3, :] += jnp.zeros((1, 128), F32) + jnp.sum(mf)
    y33 = y3.reshape(TB, K, 128)
    mm = m_ref[...][:, :, None] > 0
    hi_ref[...] = jnp.max(jnp.where(mm, y33, -_INF), axis=1)
    lo_ref[...] = jnp.min(jnp.where(mm, y33, _INF), axis=1)


def _fin_pool_body(hi_ref, lo_ref, m_ref, st_ref, ga_ref, be_ref, fe_ref):
    st = st_ref[...]
    cnt = jnp.maximum(st[2, 0], 1.0)
    mean = st[0:1, :] / cnt
    var = st[1:2, :] / cnt - mean * mean
    s = ga_ref[...] / jnp.sqrt(var + 1e-5)
    pv = jnp.where(s > 0, hi_ref[...], lo_ref[...])
    pooled = jnp.maximum((pv - mean) * s + be_ref[...], 0.0)
    anyv = jnp.max(m_ref[...], axis=1, keepdims=True) > 0
    fe_ref[...] = jnp.where(anyv, pooled, 0.0)


def _stage_a_mlp(G, U, Pg, ct, wc, mask, params):
    (_, _, g1, be1), (W2, b2, g2, be2), (W3, b3, g3, be3) = params
    NT, K = mask.shape
    TB = 128
    BR = TB * K
    nblk = NT // TB
    mf = mask.reshape(NT * K, 1)
    g_spec = pl.BlockSpec((BR, 128), lambda b: (b, 0))
    u_spec = pl.BlockSpec((TB, 128), lambda b: (b, 0))
    pg_spec = pl.BlockSpec((BR, 128), lambda b: (b, 0))
    ct_spec = pl.BlockSpec((TB, 128), lambda b: (b, 0))
    wc_spec = pl.BlockSpec((3, 128), lambda b: (0, 0))
    mf_spec = pl.BlockSpec((BR, 1), lambda b: (b, 0))
    m_spec = pl.BlockSpec((TB, K), lambda b: (b, 0))
    st_spec = pl.BlockSpec((8, 128), lambda b: (0, 0))
    w_spec = pl.BlockSpec((128, 128), lambda b: (0, 0))
    r_spec = pl.BlockSpec((1, 128), lambda b: (0, 0))
    st_shape = jax.ShapeDtypeStruct((8, 128), F32)
    y_shape = jax.ShapeDtypeStruct((NT * K, 128), F32)

    st1 = pl.pallas_call(
        functools.partial(_stats_a_body, TB, K),
        grid=(nblk,),
        in_specs=[g_spec, u_spec, pg_spec, ct_spec, wc_spec, mf_spec],
        out_specs=st_spec,
        out_shape=st_shape,
    )(G, U, Pg, ct, wc, mf)

    y2, st2 = pl.pallas_call(
        functools.partial(_l2_a_body, TB, K),
        grid=(nblk,),
        in_specs=[g_spec, u_spec, pg_spec, ct_spec, wc_spec, mf_spec, st_spec,
                  r_spec, r_spec, w_spec, r_spec],
        out_specs=[g_spec, st_spec],
        out_shape=[y_shape, st_shape],
    )(G, U, Pg, ct, wc, mf, st1, g1.reshape(1, -1), be1.reshape(1, -1), W2.T,
      b2.reshape(1, -1))

    hi, lo, st3 = pl.pallas_call(
        functools.partial(_l3_a_body, TB, K),
        grid=(nblk,),
        in_specs=[g_spec, mf_spec, m_spec, st_spec, r_spec, r_spec, w_spec,
                  r_spec],
        out_specs=[u_spec, u_spec, st_spec],
        out_shape=[jax.ShapeDtypeStruct((NT, 128), F32),
                   jax.ShapeDtypeStruct((NT, 128), F32), st_shape],
    )(y2, mf, mask, st2, g2.reshape(1, -1), be2.reshape(1, -1), W3.T,
      b3.reshape(1, -1))

    fe = pl.pallas_call(
        _fin_pool_body,
        out_shape=jax.ShapeDtypeStruct((NT, 128), F32),
    )(hi, lo, mask, st3, g3.reshape(1, -1), be3.reshape(1, -1))
    return fe


def _mlp_small_body(NT, K, D1, D2, D3, g_ref, pg_ref, ct_ref, wc_ref, b1_ref,
                    m_ref, mf_ref, ga1, be1, w2, b2, ga2, be2, w3, b3, ga3,
                    be3, fe_ref):
    m = m_ref[...]
    mf = mf_ref[...]
    cnt = jnp.maximum(jnp.sum(mf), 1.0)

    def bn(y, ga, be):
        ym = y * mf
        mean = jnp.sum(ym, axis=0, keepdims=True) / cnt
        var = jnp.sum(ym * y, axis=0, keepdims=True) / cnt - mean * mean
        return jnp.maximum((y - mean) / jnp.sqrt(var + 1e-5) * ga[...]
                           + be[...], 0.0)

    y1 = (g_ref[...] + _dpos_term(pg_ref[...][:, :16], ct_ref[...][:, :16],
                                  wc_ref[...], NT, K) + b1_ref[...])
    h1 = bn(y1, ga1, be1)
    y2 = _bdot(h1, w2[...]) + b2[...]
    h2 = bn(y2, ga2, be2)
    y3 = _bdot(h2, w3[...]) + b3[...]
    h3 = bn(y3, ga3, be3).reshape(NT, K, D3)
    hm = jnp.where(m[:, :, None] > 0, h3, -_INF)
    pooled = jnp.max(hm, axis=1)
    anyv = jnp.max(m, axis=1, keepdims=True) > 0
    fe_ref[...] = jnp.where(anyv, pooled, 0.0)


def _mlp_small(G, Pg, ct, wc, b1, mask, params):
    (_, _, g1, be1), (W2, b2, g2, be2), (W3, b3, g3, be3) = params
    NT, K = mask.shape
    D1 = G.shape[1]
    D2 = W2.shape[0]
    D3 = W3.shape[0]
    return pl.pallas_call(
        functools.partial(_mlp_small_body, NT, K, D1, D2, D3),
        out_shape=jax.ShapeDtypeStruct((NT, D3), F32),
    )(G, Pg, ct, wc, b1.reshape(1, -1), mask, mask.reshape(NT * K, 1),
      g1.reshape(1, -1), be1.reshape(1, -1), W2.T,
      b2.reshape(1, -1), g2.reshape(1, -1), be2.reshape(1, -1), W3.T,
      b3.reshape(1, -1), g3.reshape(1, -1), be3.reshape(1, -1))


def _fps_body(N, M, R, Mr, px_ref, py_ref, pz_ref, b_ref, cpx_ref, cpy_ref,
              cpz_ref, cb_ref):
    px, py, pz = px_ref[...], py_ref[...], pz_ref[...]
    bv = b_ref[...]
    iota = (lax.broadcasted_iota(jnp.int32, (R, 128), 0) * 128
            + lax.broadcasted_iota(jnp.int32, (R, 128), 1))
    iom = (lax.broadcasted_iota(jnp.int32, (Mr, 128), 0) * 128
           + lax.broadcasted_iota(jnp.int32, (Mr, 128), 1))
    px0, py0, pz0 = px[0, 0], py[0, 0], pz[0, 0]
    d0 = (px - px0) ** 2 + (py - py0) ** 2 + (pz - pz0) ** 2
    first = iom == 0
    apx = jnp.where(first, px0, 0.0)
    apy = jnp.where(first, py0, 0.0)
    apz = jnp.where(first, pz0, 0.0)
    ab = jnp.where(first, bv[0, 0], 0)

    def step(i, carry):
        dists, apx, apy, apz, ab = carry
        mx = jnp.max(dists)
        nxt = jnp.min(jnp.where(dists == mx, iota, N))
        selm = iota == nxt
        sx = jnp.sum(jnp.where(selm, px, 0.0))
        sy = jnp.sum(jnp.where(selm, py, 0.0))
        sz = jnp.sum(jnp.where(selm, pz, 0.0))
        sb = jnp.sum(jnp.where(selm, bv, 0))
        d = (px - sx) ** 2 + (py - sy) ** 2 + (pz - sz) ** 2
        dists = jnp.minimum(dists, d)
        hit = iom == i
        apx = jnp.where(hit, sx, apx)
        apy = jnp.where(hit, sy, apy)
        apz = jnp.where(hit, sz, apz)
        ab = jnp.where(hit, sb, ab)
        return dists, apx, apy, apz, ab

    _, apx, apy, apz, ab = lax.fori_loop(1, M, step, (d0, apx, apy, apz, ab))
    cpx_ref[...] = apx
    cpy_ref[...] = apy
    cpz_ref[...] = apz
    cb_ref[...] = ab


def _fps(pos8, b16, M):
    N = pos8.shape[0]
    R = N // 128
    Mr = M // 128
    pcs = [pos8[:, c].reshape(R, 128) for c in range(3)]
    o = jax.ShapeDtypeStruct((Mr, 128), F32)
    return pl.pallas_call(
        functools.partial(_fps_body, N, M, R, Mr),
        out_shape=[o, o, o, jax.ShapeDtypeStruct((Mr, 128), jnp.int32)],
    )(*pcs, b16)


def kernel(f1, pos1, batch1, f2, pos2, batch2, params_fe, params_sc1,
           params_sc2):
    N = f1.shape[0]
    pos1p = jnp.pad(pos1, ((0, 0), (0, 5)))
    pos2p = jnp.pad(pos2, ((0, 0), (0, 5)))
    pos1t = jnp.pad(pos1, ((0, 0), (0, 125)))
    pos2t = jnp.pad(pos2, ((0, 0), (0, 125)))

    W1, b1, _, _ = params_fe[0]
    WaT = W1[:, :128].T
    WbT = W1[:, 128:256].T
    WcT = W1[:, 256:259].T
    zrow = jnp.zeros((1, 128), F32)

    idx1, m1 = _knn(pos1p, pos2p, 64, 25.0, 256)
    U1 = _matmul(f1, WaT, b1.reshape(1, -1))
    V1 = _matmul(f2, WbT, zrow)
    G1 = _gather_rows(V1, idx1.reshape(-1))
    P1 = _gather_rows(pos2t, idx1.reshape(-1))
    fe1 = _stage_a_mlp(G1, U1, P1, pos1t, WcT, m1, params_fe)

    W1s, b1s, _, _ = params_sc1[0]
    WfT2 = W1s[:, :128].T
    WpT2 = W1s[:, 128:131].T
    cx2, cy2, cz2, cb2 = _fps(pos1p, batch1.reshape(N // 128, 128), N // 4)
    cpos2 = jnp.stack([cx2.reshape(-1), cy2.reshape(-1), cz2.reshape(-1)],
                      axis=1)
    cpos2p = jnp.pad(cpos2, ((0, 0), (0, 5)))
    cpos2t = jnp.pad(cpos2, ((0, 0), (0, 125)))
    idx2, m2 = _knn(cpos2p, pos1p, 8, 4.0, N // 4)
    V2 = _matmul(fe1, WfT2, zrow)
    G2 = _gather_rows(V2, idx2.reshape(-1))
    P2 = _gather_rows(pos1t, idx2.reshape(-1))
    fe2 = _mlp_small(G2, P2, cpos2t, WpT2, b1s, m2, params_sc1)

    W1t, b1t, _, _ = params_sc2[0]
    WfT3 = W1t[:, :256].T
    WpT3 = W1t[:, 256:259].T
    cx3, cy3, cz3, cb3 = _fps(cpos2p, cb2.reshape(N // 512, 128), N // 16)
    cpos3 = jnp.stack([cx3.reshape(-1), cy3.reshape(-1), cz3.reshape(-1)],
                      axis=1)
    cpos3p = jnp.pad(cpos3, ((0, 0), (0, 5)))
    cpos3t = jnp.pad(cpos3, ((0, 0), (0, 125)))
    idx3, m3 = _knn(cpos3p, cpos2p, 8, 16.0, N // 16)
    V3 = _matmul(fe2, WfT3, jnp.zeros((1, 256), F32))
    G3 = _gather_rows(V3, idx3.reshape(-1))
    P3 = _gather_rows(cpos2t, idx3.reshape(-1))
    fe3 = _mlp_small(G3, P3, cpos3t, WpT3, b1t, m3, params_sc2)

    return ((fe1, pos1, batch1),
            (fe2, cpos2, cb2.reshape(-1)),
            (fe3, cpos3, cb3.reshape(-1)))
